# Initial kernel scaffold; baseline (speedup 1.0000x reference)
import jax
import jax.numpy as jnp
from jax.experimental import pallas as pl


def kernel(feat, edge_index, W1, b1, W2, b2):
    raise NotImplementedError("write your pallas kernel here")


# speedup vs baseline: 10.7955x; 10.7955x over previous
import functools

import jax
import jax.numpy as jnp
from jax import lax
from jax.experimental import pallas as pl
from jax.experimental.pallas import tpu as pltpu
from jax.experimental.pallas import tpu_sc as plsc

N = 10000
D = 128
D2 = D // 2
E = 320000
P = 10240
LAM = 0.9
ALP = 1.0 / (LAM + 1.0)
PROP_STEP = 8

NC, NS = 2, 16
NW = NC * NS
K = 128
CHD = 80
CH = 160
EP = NS * CH * K
NBUF = 4
RPT = P // NS
RB = 640

_mesh = plsc.VectorSubcoreMesh(core_axis_name="c", subcore_axis_name="s")


@functools.partial(
    pl.kernel,
    out_type=jax.ShapeDtypeStruct((NC, P), jnp.float32),
    mesh=_mesh,
    compiler_params=pltpu.CompilerParams(use_tc_tiling_on_sc=False),
    scratch_types=[
        pltpu.VMEM((CHD, K), jnp.int32),
        pltpu.VMEM((K,), jnp.float32),
        pltpu.VMEM((RPT,), jnp.float32),
        pltpu.VMEM_SHARED((P,), jnp.float32),
    ],
)
def _deg_kernel(dst_hbm, out_hbm, dst_v, ones_v, zbuf, dacc):
    cid = lax.axis_index("c")
    sid = lax.axis_index("s")
    wid = sid * NC + cid
    pltpu.sync_copy(dst_hbm.at[wid], dst_v)

    def _z(i, c):
        zbuf[pl.ds(i * 16, 16)] = jnp.zeros((16,), jnp.float32)
        return c
    lax.fori_loop(0, RPT // 16, _z, 0)

    def _o(i, c):
        ones_v[pl.ds(i * 16, 16)] = jnp.ones((16,), jnp.float32)
        return c
    lax.fori_loop(0, K // 16, _o, 0)

    pltpu.sync_copy(zbuf, dacc.at[pl.ds(sid * RPT, RPT)])
    plsc.subcore_barrier()

    def _s(j, c):
        pltpu.sync_copy(ones_v, dacc.at[dst_v.at[j]], add=True)
        return c
    lax.fori_loop(0, CHD, _s, 0)

    plsc.subcore_barrier()
    pltpu.sync_copy(dacc.at[pl.ds(sid * RPT, RPT)],
                    out_hbm.at[cid, pl.ds(sid * RPT, RPT)])


@functools.partial(
    pl.kernel,
    out_type=[jax.ShapeDtypeStruct((P, D2), jnp.float32),
              jax.ShapeDtypeStruct((P, D2), jnp.float32)],
    mesh=_mesh,
    compiler_params=pltpu.CompilerParams(use_tc_tiling_on_sc=False),
    scratch_types=[
        pltpu.VMEM((CH, K), jnp.int32),
        pltpu.VMEM((CH, K), jnp.int32),
        pltpu.VMEM((NBUF, K, D2), jnp.float32),
        pltpu.VMEM_SHARED((P, D2), jnp.float32),
        pltpu.SemaphoreType.DMA((NBUF,)),
        pltpu.SemaphoreType.DMA((NBUF,)),
    ],
)
def _scatter_kernel(s0_hbm, s1_hbm, src_hbm, dst_hbm, a0_hbm, a1_hbm,
                    src_v, dst_v, rowbuf, acc, gsem, ssem):
    cid = lax.axis_index("c")
    sid = lax.axis_index("s")
    pltpu.sync_copy(src_hbm.at[sid], src_v)
    pltpu.sync_copy(dst_hbm.at[sid], dst_v)

    def _z(r, c):
        for cc in range(D2 // 16):
            rowbuf[0, r, pl.ds(cc * 16, 16)] = jnp.zeros((16,), jnp.float32)
        return c
    lax.fori_loop(0, K, _z, 0)
    for t in range(RPT // K):
        pltpu.sync_copy(rowbuf.at[0], acc.at[pl.ds(sid * RPT + t * K, K)])
    plsc.subcore_barrier()

    def _ring(s_hbm):
        for b in range(NBUF):
            pltpu.async_copy(s_hbm.at[src_v.at[b]], rowbuf.at[b], gsem.at[b])

        def _body(it, c):
            j0 = it * NBUF
            for b in range(NBUF):
                j = j0 + b
                pltpu.make_async_copy(s_hbm.at[src_v.at[j]], rowbuf.at[b],
                                      gsem.at[b]).wait()
                pltpu.async_copy(rowbuf.at[b], acc.at[dst_v.at[j]],
                                 ssem.at[b], add=True)
                pltpu.make_async_copy(rowbuf.at[b], acc.at[dst_v.at[j]],
                                      ssem.at[b]).wait()
                pltpu.async_copy(s_hbm.at[src_v.at[j + NBUF]], rowbuf.at[b],
                                 gsem.at[b])
            return c
        lax.fori_loop(0, CH // NBUF - 1, _body, 0)

        for b in range(NBUF):
            j = CH - NBUF + b
            pltpu.make_async_copy(s_hbm.at[src_v.at[j]], rowbuf.at[b],
                                  gsem.at[b]).wait()
            pltpu.async_copy(rowbuf.at[b], acc.at[dst_v.at[j]],
                             ssem.at[b], add=True)
            pltpu.make_async_copy(rowbuf.at[b], acc.at[dst_v.at[j]],
                                  ssem.at[b]).wait()

    @pl.when(cid == 0)
    def _():
        _ring(s0_hbm)

    @pl.when(cid != 0)
    def _():
        _ring(s1_hbm)

    plsc.subcore_barrier()

    @pl.when(cid == 0)
    def _():
        pltpu.sync_copy(acc.at[pl.ds(sid * RPT, RPT)],
                        a0_hbm.at[pl.ds(sid * RPT, RPT)])

    @pl.when(cid != 0)
    def _():
        pltpu.sync_copy(acc.at[pl.ds(sid * RPT, RPT)],
                        a1_hbm.at[pl.ds(sid * RPT, RPT)])


def _mm_body(x_ref, w_ref, b_ref, o_ref, *, relu):
    x = x_ref[...]
    if relu:
        x = jnp.maximum(x, 0.0)
    o_ref[...] = lax.dot_general(x, w_ref[...], (((1,), (1,)), ((), ())),
                                 preferred_element_type=jnp.float32) + b_ref[...]


def _mm(x, w, b, relu):
    return pl.pallas_call(
        functools.partial(_mm_body, relu=relu),
        grid=(P // RB,),
        in_specs=[
            pl.BlockSpec((RB, D), lambda i: (i, 0)),
            pl.BlockSpec((D, D), lambda i: (0, 0)),
            pl.BlockSpec((1, D), lambda i: (0, 0)),
        ],
        out_specs=pl.BlockSpec((RB, D), lambda i: (i, 0)),
        out_shape=jax.ShapeDtypeStruct((P, D), jnp.float32),
    )(x, w, b)


def _prep_body(x_ref, dmbh_ref, dmb1_ref, c_ref, s0_ref, s1_ref):
    x = x_ref[...]
    c_ref[...] = ALP * x * dmb1_ref[...]
    s = x * dmbh_ref[...]
    s0_ref[...] = s[:, :D2]
    s1_ref[...] = s[:, D2:]


def _prep(x, dmbh, dmb1):
    return pl.pallas_call(
        _prep_body,
        grid=(P // RB,),
        in_specs=[
            pl.BlockSpec((RB, D), lambda i: (i, 0)),
            pl.BlockSpec((RB, 1), lambda i: (i, 0)),
            pl.BlockSpec((RB, 1), lambda i: (i, 0)),
        ],
        out_specs=[
            pl.BlockSpec((RB, D), lambda i: (i, 0)),
            pl.BlockSpec((RB, D2), lambda i: (i, 0)),
            pl.BlockSpec((RB, D2), lambda i: (i, 0)),
        ],
        out_shape=[
            jax.ShapeDtypeStruct((P, D), jnp.float32),
            jax.ShapeDtypeStruct((P, D2), jnp.float32),
            jax.ShapeDtypeStruct((P, D2), jnp.float32),
        ],
    )(x, dmbh, dmb1)


def _update_body(y_ref, a0_ref, a1_ref, c_ref, dmbh_ref,
                 yo_ref, s0_ref, s1_ref):
    dm = dmbh_ref[...]
    t = jnp.concatenate([a0_ref[...], a1_ref[...]], axis=1)
    y = (1.0 - ALP) * y_ref[...] + (ALP * LAM) * (t * dm) + c_ref[...]
    yo_ref[...] = y
    s = y * dm
    s0_ref[...] = s[:, :D2]
    s1_ref[...] = s[:, D2:]


def _update(y, a0, a1, c, dmbh):
    return pl.pallas_call(
        _update_body,
        grid=(P // RB,),
        in_specs=[
            pl.BlockSpec((RB, D), lambda i: (i, 0)),
            pl.BlockSpec((RB, D2), lambda i: (i, 0)),
            pl.BlockSpec((RB, D2), lambda i: (i, 0)),
            pl.BlockSpec((RB, D), lambda i: (i, 0)),
            pl.BlockSpec((RB, 1), lambda i: (i, 0)),
        ],
        out_specs=[
            pl.BlockSpec((RB, D), lambda i: (i, 0)),
            pl.BlockSpec((RB, D2), lambda i: (i, 0)),
            pl.BlockSpec((RB, D2), lambda i: (i, 0)),
        ],
        out_shape=[
            jax.ShapeDtypeStruct((P, D), jnp.float32),
            jax.ShapeDtypeStruct((P, D2), jnp.float32),
            jax.ShapeDtypeStruct((P, D2), jnp.float32),
        ],
    )(y, a0, a1, c, dmbh)


def kernel(feat, edge_index, W1, b1, W2, b2):
    src = edge_index[0].astype(jnp.int32)
    dst = edge_index[1].astype(jnp.int32)
    fill = jnp.arange(EP - E, dtype=jnp.int32)
    src_p = jnp.concatenate([src, fill % N])
    dst_p = jnp.concatenate([dst, N + fill % (P - N)])
    src_w = src_p.reshape(NS, CH, K)
    dst_w = dst_p.reshape(NS, CH, K)
    dst_d = dst_p.reshape(NW, CHD, K)
    feat_p = jnp.pad(feat, ((0, P - N), (0, 0)))
    b1r = b1.reshape(1, D)
    b2r = b2.reshape(1, D)

    X = _mm(feat_p, W1, b1r, relu=False)
    deg2 = _deg_kernel(dst_d)
    deg = deg2[0] + deg2[1]
    db = LAM * deg + (1.0 - LAM)
    m = jnp.arange(P) < N
    dmbh = jnp.where(m, lax.rsqrt(db), 0.0)[:, None]
    dmb1 = jnp.where(m, 1.0 / db, 0.0)[:, None]
    C, S0, S1 = _prep(X, dmbh, dmb1)

    Y = X
    for _ in range(PROP_STEP):
        A0, A1 = _scatter_kernel(S0, S1, src_w, dst_w)
        Y, S0, S1 = _update(Y, A0, A1, C, dmbh)

    out = _mm(Y, W2, b2r, relu=True)
    return out[:N]
